# baseline (speedup 1.0000x reference)
import jax
import jax.numpy as jnp
from jax import lax
from jax.experimental import pallas as pl
from jax.experimental.pallas import tpu as pltpu
from jax.experimental.pallas import tpu_sc as plsc

NUM_VOCAB = 1000000
MAXLEN = 200
EMBED_DIM = 32
BATCH = 4096
SEQ = 200

NC = 2
NS = 16
NW = NC * NS
BT = BATCH // 128
NSLAB = SEQ * BT
SPW = NSLAB // NW
DG = EMBED_DIM // 8
LANES = 16


def _emb_body(x_hbm, tok_hbm, pos_hbm, out_hbm,
              idx_v, pos_v, pb_v, g_v, t_v, sem):
    c = lax.axis_index("c")
    s_ax = lax.axis_index("s")
    wid = s_ax * NC + c
    slab0 = wid * SPW

    pltpu.sync_copy(pos_hbm, pos_v)

    iota = lax.iota(jnp.int32, LANES)
    row_js = [iota + (j * LANES) for j in range(128 // LANES)]

    @pl.loop(0, SPW)
    def _(i):
        slab = slab0 + i
        s = slab // BT
        bt = slab % BT

        @pl.when(jnp.logical_or(bt == 0, i == 0))
        def _():
            pltpu.sync_copy(x_hbm.at[s], idx_v)
            s16 = jnp.full((LANES,), s, jnp.int32)

            @pl.loop(0, EMBED_DIM)
            def _(d):
                d16 = jnp.full((LANES,), d, jnp.int32)
                pb_v[d, :] = plsc.load_gather(pos_v, [s16, d16])

        pltpu.async_copy(tok_hbm.at[idx_v.at[bt]], g_v, sem).wait()

        @pl.loop(0, EMBED_DIM)
        def _(d):
            d16 = jnp.full((LANES,), d, jnp.int32)
            pb = pb_v[d, :]
            for j in range(128 // LANES):
                t_v[d, pl.ds(j * LANES, LANES)] = (
                    plsc.load_gather(g_v, [row_js[j], d16]) + pb)

        for dg in range(DG):
            pltpu.sync_copy(t_v.at[pl.ds(dg * 8, 8)],
                            out_hbm.at[s * DG + dg, bt])


def kernel(x, token_table, position_table):
    xt = x.T.reshape(SEQ, BT, 128).astype(jnp.int32)
    mesh = plsc.VectorSubcoreMesh(core_axis_name="c", subcore_axis_name="s")
    run = pl.kernel(
        _emb_body,
        out_type=jax.ShapeDtypeStruct((SEQ * DG, BT, 8, 128), jnp.float32),
        mesh=mesh,
        scratch_types=[
            pltpu.VMEM((BT, 128), jnp.int32),
            pltpu.VMEM((MAXLEN, EMBED_DIM), jnp.float32),
            pltpu.VMEM((EMBED_DIM, LANES), jnp.float32),
            pltpu.VMEM((128, EMBED_DIM), jnp.float32),
            pltpu.VMEM((EMBED_DIM, 128), jnp.float32),
            pltpu.SemaphoreType.DMA,
        ],
        compiler_params=pltpu.CompilerParams(
            use_tc_tiling_on_sc=False, needs_layout_passes=False),
    )
    p5 = run(xt, token_table, position_table)
    return (p5.reshape(SEQ, DG, BT, 8, 128)
            .transpose(2, 4, 0, 1, 3)
            .reshape(BATCH, SEQ, EMBED_DIM))


# speedup vs baseline: 1.7713x; 1.7713x over previous
import jax
import jax.numpy as jnp
from jax import lax
from jax.experimental import pallas as pl
from jax.experimental.pallas import tpu as pltpu
from jax.experimental.pallas import tpu_sc as plsc

NUM_VOCAB = 1000000
MAXLEN = 200
EMBED_DIM = 32
BATCH = 4096
SEQ = 200

NC = 2
NS = 16
NW = NC * NS
BT = BATCH // 128
NSLAB = SEQ * BT
SPW = NSLAB // NW
DG = EMBED_DIM // 8
LANES = 16
NBUF = 4


def _gather_body(x_hbm, tok_hbm, pos_hbm, out_hbm,
                 idx_v, pos_v, gbufs, wbufs, gsems, wsems):
    c = lax.axis_index("c")
    s_ax = lax.axis_index("s")
    wid = s_ax * NC + c
    slab0 = wid * SPW

    pltpu.sync_copy(pos_hbm, pos_v)
    pltpu.sync_copy(x_hbm.at[pl.ds(slab0, SPW)], idx_v)

    def start_gather(k, b):
        pltpu.async_copy(tok_hbm.at[idx_v.at[k]], gbufs[b], gsems[b])

    def wait_gather(b):
        pltpu.make_async_copy(
            tok_hbm.at[pl.ds(0, 128)], gbufs[b], gsems[b]).wait()

    def start_wb(k, b):
        pltpu.async_copy(wbufs[b], out_hbm.at[slab0 + k], wsems[b])

    def wait_wb(b):
        pltpu.make_async_copy(wbufs[b], out_hbm.at[0], wsems[b]).wait()

    for b in range(NBUF):
        start_gather(b, b)

    @pl.loop(0, SPW, step=NBUF)
    def _(g):
        for b in range(NBUF):
            k = g + b
            s = (slab0 + k) // BT
            wait_gather(b)

            @pl.when(g > 0)
            def _():
                wait_wb(b)

            gbuf, wbuf = gbufs[b], wbufs[b]
            p0 = pos_v[s, pl.ds(0, LANES)]
            p1 = pos_v[s, pl.ds(LANES, LANES)]

            @pl.loop(0, 128)
            def _(r):
                wbuf[r, pl.ds(0, LANES)] = gbuf[r, pl.ds(0, LANES)] + p0
                wbuf[r, pl.ds(LANES, LANES)] = (
                    gbuf[r, pl.ds(LANES, LANES)] + p1)

            @pl.when(g < SPW - NBUF)
            def _():
                start_gather(k + NBUF, b)

            start_wb(k, b)

    for b in range(NBUF):
        wait_wb(b)


def _relayout_body(in_ref, out_ref):
    t = in_ref[...].T
    for dg in range(DG):
        for bt in range(BT):
            out_ref[0, dg, bt] = jnp.concatenate(
                [t[m * 32 + dg * 8:m * 32 + dg * 8 + 8,
                   bt * 32:(bt + 1) * 32] for m in range(4)],
                axis=-1)


def kernel(x, token_table, position_table):
    xt = (x.T.reshape(SEQ, BT, 4, 32)
          .transpose(0, 1, 3, 2)
          .reshape(NSLAB, 128)
          .astype(jnp.int32))
    mesh = plsc.VectorSubcoreMesh(core_axis_name="c", subcore_axis_name="s")
    gather = pl.kernel(
        _gather_body,
        out_type=jax.ShapeDtypeStruct((NSLAB, 128, EMBED_DIM), jnp.float32),
        mesh=mesh,
        scratch_types=[
            pltpu.VMEM((SPW, 128), jnp.int32),
            pltpu.VMEM((MAXLEN, EMBED_DIM), jnp.float32),
            [pltpu.VMEM((128, EMBED_DIM), jnp.float32) for _ in range(NBUF)],
            [pltpu.VMEM((128, EMBED_DIM), jnp.float32) for _ in range(NBUF)],
            [pltpu.SemaphoreType.DMA for _ in range(NBUF)],
            [pltpu.SemaphoreType.DMA for _ in range(NBUF)],
        ],
        compiler_params=pltpu.CompilerParams(use_tc_tiling_on_sc=False),
    )
    inter = gather(xt, token_table, position_table)
    inter2 = inter.reshape(SEQ * BATCH * EMBED_DIM // 128, 128)

    p5 = pl.pallas_call(
        _relayout_body,
        grid=(SEQ,),
        in_specs=[pl.BlockSpec((BATCH * EMBED_DIM // 128, 128),
                               lambda s: (s, 0))],
        out_specs=pl.BlockSpec((1, DG, BT, 8, 128),
                               lambda s: (s, 0, 0, 0, 0)),
        out_shape=jax.ShapeDtypeStruct((SEQ, DG, BT, 8, 128), jnp.float32),
    )(inter2)

    return (p5.transpose(2, 4, 0, 1, 3).reshape(BATCH, SEQ, EMBED_DIM))


# speedup vs baseline: 2.2471x; 1.2686x over previous
import jax
import jax.numpy as jnp
from jax import lax
from jax.experimental import pallas as pl
from jax.experimental.pallas import tpu as pltpu
from jax.experimental.pallas import tpu_sc as plsc

NUM_VOCAB = 1000000
MAXLEN = 200
EMBED_DIM = 32
BATCH = 4096
SEQ = 200

NC = 2
NS = 16
NW = NC * NS
BT = BATCH // 128
NSLAB = SEQ * BT
SPW = NSLAB // NW
DG = EMBED_DIM // 8
LANES = 16
NBUF = 4


def _gather_body(x_hbm, tok_hbm, pos_hbm, out_hbm,
                 idx_v, pos_v, gbufs, wbufs, gsems, wsems):
    c = lax.axis_index("c")
    s_ax = lax.axis_index("s")
    wid = s_ax * NC + c
    slab0 = wid * SPW

    pltpu.sync_copy(pos_hbm, pos_v)
    pltpu.sync_copy(x_hbm.at[pl.ds(slab0, SPW)], idx_v)

    def start_gather(k, b):
        pltpu.async_copy(tok_hbm.at[idx_v.at[k]], gbufs[b], gsems[b])

    def wait_gather(b):
        pltpu.make_async_copy(
            tok_hbm.at[pl.ds(0, 128)], gbufs[b], gsems[b]).wait()

    def start_wb(k, b):
        pltpu.async_copy(wbufs[b], out_hbm.at[slab0 + k], wsems[b])

    def wait_wb(b):
        pltpu.make_async_copy(wbufs[b], out_hbm.at[0], wsems[b]).wait()

    for b in range(NBUF):
        start_gather(b, b)

    @pl.loop(0, SPW, step=NBUF)
    def _(g):
        for b in range(NBUF):
            k = g + b
            s = (slab0 + k) // BT
            wait_gather(b)

            @pl.when(g > 0)
            def _():
                wait_wb(b)

            gbuf, wbuf = gbufs[b], wbufs[b]
            p0 = pos_v[s, pl.ds(0, LANES)]
            p1 = pos_v[s, pl.ds(LANES, LANES)]

            @pl.loop(0, 128)
            def _(r):
                wbuf[r, pl.ds(0, LANES)] = gbuf[r, pl.ds(0, LANES)] + p0
                wbuf[r, pl.ds(LANES, LANES)] = (
                    gbuf[r, pl.ds(LANES, LANES)] + p1)

            @pl.when(g < SPW - NBUF)
            def _():
                start_gather(k + NBUF, b)

            start_wb(k, b)

    for b in range(NBUF):
        wait_wb(b)


def _conv_body(in_ref, out_ref):
    t = in_ref[...].T
    out_ref[...] = jnp.concatenate(
        [t[m * 1024:(m + 1) * 1024, :] for m in range(4)], axis=-1)


def _relayout_body(in_ref, out_ref):
    t = in_ref[...].T
    for dg in range(DG):
        for bt in range(BT):
            out_ref[0, dg, bt] = jnp.concatenate(
                [t[m * 32 + dg * 8:m * 32 + dg * 8 + 8,
                   bt * 32:(bt + 1) * 32] for m in range(4)],
                axis=-1)


VBLK = 4096
NVBLK = -(-NUM_VOCAB // VBLK)


def kernel(x, token_table, position_table):
    tok_lin2 = pl.pallas_call(
        _conv_body,
        grid=(NVBLK,),
        in_specs=[pl.BlockSpec((EMBED_DIM, VBLK), lambda i: (0, i))],
        out_specs=pl.BlockSpec((VBLK // 4, 128), lambda i: (i, 0)),
        out_shape=jax.ShapeDtypeStruct((NVBLK * VBLK // 4, 128), jnp.float32),
    )(token_table.T)
    tok_lin = tok_lin2.reshape(NVBLK * VBLK, EMBED_DIM)

    x = x.astype(jnp.int32)
    q = x % VBLK
    xr = (x - q) + 4 * (q % (VBLK // 4)) + q // (VBLK // 4)

    xt = (xr.T.reshape(SEQ, BT, 4, 32)
          .transpose(0, 1, 3, 2)
          .reshape(NSLAB, 128))
    mesh = plsc.VectorSubcoreMesh(core_axis_name="c", subcore_axis_name="s")
    gather = pl.kernel(
        _gather_body,
        out_type=jax.ShapeDtypeStruct((NSLAB, 128, EMBED_DIM), jnp.float32),
        mesh=mesh,
        scratch_types=[
            pltpu.VMEM((SPW, 128), jnp.int32),
            pltpu.VMEM((MAXLEN, EMBED_DIM), jnp.float32),
            [pltpu.VMEM((128, EMBED_DIM), jnp.float32) for _ in range(NBUF)],
            [pltpu.VMEM((128, EMBED_DIM), jnp.float32) for _ in range(NBUF)],
            [pltpu.SemaphoreType.DMA for _ in range(NBUF)],
            [pltpu.SemaphoreType.DMA for _ in range(NBUF)],
        ],
        compiler_params=pltpu.CompilerParams(use_tc_tiling_on_sc=False),
    )
    inter = gather(xt, tok_lin, position_table)
    inter2 = inter.reshape(SEQ * BATCH * EMBED_DIM // 128, 128)

    p5 = pl.pallas_call(
        _relayout_body,
        grid=(SEQ,),
        in_specs=[pl.BlockSpec((BATCH * EMBED_DIM // 128, 128),
                               lambda s: (s, 0))],
        out_specs=pl.BlockSpec((1, DG, BT, 8, 128),
                               lambda s: (s, 0, 0, 0, 0)),
        out_shape=jax.ShapeDtypeStruct((SEQ, DG, BT, 8, 128), jnp.float32),
    )(inter2)

    return (p5.transpose(2, 4, 0, 1, 3).reshape(BATCH, SEQ, EMBED_DIM))


# speedup vs baseline: 2.8041x; 1.2479x over previous
import jax
import jax.numpy as jnp
from jax import lax
from jax.experimental import pallas as pl
from jax.experimental.pallas import tpu as pltpu
from jax.experimental.pallas import tpu_sc as plsc

NUM_VOCAB = 1000000
MAXLEN = 200
EMBED_DIM = 32
BATCH = 4096
SEQ = 200

NC = 2
NS = 16
NW = NC * NS
BT = BATCH // 128
NSLAB = SEQ * BT
SPW = NSLAB // NW
DG = EMBED_DIM // 8
LANES = 16
NBUF = 4


def _gather_body(x_hbm, tok_hbm, pos_hbm, out_hbm,
                 idx_v, pos_v, gbufs, wbufs, gsems, wsems):
    c = lax.axis_index("c")
    s_ax = lax.axis_index("s")
    wid = s_ax * NC + c
    slab0 = wid * SPW

    pltpu.sync_copy(pos_hbm, pos_v)
    pltpu.sync_copy(x_hbm.at[pl.ds(slab0, SPW)], idx_v)

    def start_gather(k, b):
        pltpu.async_copy(tok_hbm.at[idx_v.at[k]], gbufs[b], gsems[b])

    def wait_gather(b):
        pltpu.make_async_copy(
            tok_hbm.at[pl.ds(0, 128)], gbufs[b], gsems[b]).wait()

    def start_wb(k, b):
        pltpu.async_copy(wbufs[b], out_hbm.at[slab0 + k], wsems[b])

    def wait_wb(b):
        pltpu.make_async_copy(wbufs[b], out_hbm.at[0], wsems[b]).wait()

    for b in range(NBUF):
        start_gather(b, b)

    @pl.loop(0, SPW, step=NBUF)
    def _(g):
        for b in range(NBUF):
            k = g + b
            s = (slab0 + k) // BT
            wait_gather(b)

            @pl.when(g > 0)
            def _():
                wait_wb(b)

            gbuf, wbuf = gbufs[b], wbufs[b]
            p0 = pos_v[s, pl.ds(0, LANES)]
            p1 = pos_v[s, pl.ds(LANES, LANES)]

            @pl.loop(0, 128)
            def _(r):
                wbuf[r, pl.ds(0, LANES)] = gbuf[r, pl.ds(0, LANES)] + p0
                wbuf[r, pl.ds(LANES, LANES)] = (
                    gbuf[r, pl.ds(LANES, LANES)] + p1)

            @pl.when(g < SPW - NBUF)
            def _():
                start_gather(k + NBUF, b)

            start_wb(k, b)

    for b in range(NBUF):
        wait_wb(b)


VBLK = 16384
NVBLK = -(-NUM_VOCAB // VBLK)
QB = VBLK // 4
SBLK = 4


def _conv_body(in_ref, out_ref):
    t = in_ref[...].T
    out_ref[...] = jnp.concatenate(
        [t[m * QB:(m + 1) * QB, :] for m in range(4)], axis=-1)


def _relayout_body(in_ref, out_ref):
    t = in_ref[...].T
    for sl in range(SBLK):
        for dg in range(DG):
            for bt in range(BT):
                c = sl * 1024 + bt * 32
                out_ref[sl, dg, bt] = jnp.concatenate(
                    [t[m * 32 + dg * 8:m * 32 + dg * 8 + 8,
                       c:c + 32] for m in range(4)],
                    axis=-1)


def kernel(x, token_table, position_table):
    tok_lin2 = pl.pallas_call(
        _conv_body,
        grid=(NVBLK,),
        in_specs=[pl.BlockSpec((EMBED_DIM, VBLK), lambda i: (0, i))],
        out_specs=pl.BlockSpec((VBLK // 4, 128), lambda i: (i, 0)),
        out_shape=jax.ShapeDtypeStruct((NVBLK * VBLK // 4, 128), jnp.float32),
    )(token_table.T)
    tok_lin = tok_lin2.reshape(NVBLK * VBLK, EMBED_DIM)

    x = x.astype(jnp.int32)
    q = x % VBLK
    xr = (x - q) + 4 * (q % QB) + q // QB

    xt = (xr.T.reshape(SEQ, BT, 4, 32)
          .transpose(0, 1, 3, 2)
          .reshape(NSLAB, 128))
    mesh = plsc.VectorSubcoreMesh(core_axis_name="c", subcore_axis_name="s")
    gather = pl.kernel(
        _gather_body,
        out_type=jax.ShapeDtypeStruct((NSLAB, 128, EMBED_DIM), jnp.float32),
        mesh=mesh,
        scratch_types=[
            pltpu.VMEM((SPW, 128), jnp.int32),
            pltpu.VMEM((MAXLEN, EMBED_DIM), jnp.float32),
            [pltpu.VMEM((128, EMBED_DIM), jnp.float32) for _ in range(NBUF)],
            [pltpu.VMEM((128, EMBED_DIM), jnp.float32) for _ in range(NBUF)],
            [pltpu.SemaphoreType.DMA for _ in range(NBUF)],
            [pltpu.SemaphoreType.DMA for _ in range(NBUF)],
        ],
        compiler_params=pltpu.CompilerParams(use_tc_tiling_on_sc=False),
    )
    inter = gather(xt, tok_lin, position_table)
    inter2 = inter.reshape(SEQ * BATCH * EMBED_DIM // 128, 128)

    p5 = pl.pallas_call(
        _relayout_body,
        grid=(SEQ // SBLK,),
        in_specs=[pl.BlockSpec((SBLK * BATCH * EMBED_DIM // 128, 128),
                               lambda s: (s, 0))],
        out_specs=pl.BlockSpec((SBLK, DG, BT, 8, 128),
                               lambda s: (s, 0, 0, 0, 0)),
        out_shape=jax.ShapeDtypeStruct((SEQ, DG, BT, 8, 128), jnp.float32),
    )(inter2)

    return (p5.transpose(2, 4, 0, 1, 3).reshape(BATCH, SEQ, EMBED_DIM))


# speedup vs baseline: 2.8496x; 1.0162x over previous
import jax
import jax.numpy as jnp
from jax import lax
from jax.experimental import pallas as pl
from jax.experimental.pallas import tpu as pltpu
from jax.experimental.pallas import tpu_sc as plsc

NUM_VOCAB = 1000000
MAXLEN = 200
EMBED_DIM = 32
BATCH = 4096
SEQ = 200

NC = 2
NS = 16
NW = NC * NS
BT = BATCH // 128
NSLAB = SEQ * BT
SPW = NSLAB // NW
DG = EMBED_DIM // 8
LANES = 16
NBUF = 4


def _gather_body(x_hbm, tok_hbm, pos_hbm, out_hbm,
                 idx_v, pos_v, gbufs, wbufs, gsems, wsems):
    c = lax.axis_index("c")
    s_ax = lax.axis_index("s")
    wid = s_ax * NC + c
    slab0 = wid * SPW

    pltpu.sync_copy(pos_hbm, pos_v)
    pltpu.sync_copy(x_hbm.at[pl.ds(slab0, SPW)], idx_v)

    def start_gather(k, b):
        pltpu.async_copy(tok_hbm.at[idx_v.at[k]], gbufs[b], gsems[b])

    def wait_gather(b):
        pltpu.make_async_copy(
            tok_hbm.at[pl.ds(0, 128)], gbufs[b], gsems[b]).wait()

    def start_wb(k, b):
        pltpu.async_copy(wbufs[b], out_hbm.at[slab0 + k], wsems[b])

    def wait_wb(b):
        pltpu.make_async_copy(wbufs[b], out_hbm.at[0], wsems[b]).wait()

    for b in range(NBUF):
        start_gather(b, b)

    @pl.loop(0, SPW, step=NBUF)
    def _(g):
        for b in range(NBUF):
            k = g + b
            s = (slab0 + k) // BT
            wait_gather(b)

            @pl.when(g > 0)
            def _():
                wait_wb(b)

            gbuf, wbuf = gbufs[b], wbufs[b]
            p0 = pos_v[s, pl.ds(0, LANES)]
            p1 = pos_v[s, pl.ds(LANES, LANES)]

            @pl.loop(0, 128)
            def _(r):
                wbuf[r, pl.ds(0, LANES)] = gbuf[r, pl.ds(0, LANES)] + p0
                wbuf[r, pl.ds(LANES, LANES)] = (
                    gbuf[r, pl.ds(LANES, LANES)] + p1)

            @pl.when(g < SPW - NBUF)
            def _():
                start_gather(k + NBUF, b)

            start_wb(k, b)

    for b in range(NBUF):
        wait_wb(b)


VBLK = 32768
NVBLK = -(-NUM_VOCAB // VBLK)
QB = VBLK // 4
SBLK = 8


def _conv_body(in_ref, out_ref):
    t = in_ref[...].T
    out_ref[...] = jnp.concatenate(
        [t[m * QB:(m + 1) * QB, :] for m in range(4)], axis=-1)


def _relayout_body(in_ref, out_ref):
    t = in_ref[...].T
    for sl in range(SBLK):
        for dg in range(DG):
            for bt in range(BT):
                c = sl * 1024 + bt * 32
                out_ref[sl, dg, bt] = jnp.concatenate(
                    [t[m * 32 + dg * 8:m * 32 + dg * 8 + 8,
                       c:c + 32] for m in range(4)],
                    axis=-1)


def kernel(x, token_table, position_table):
    tok_lin2 = pl.pallas_call(
        _conv_body,
        grid=(NVBLK,),
        in_specs=[pl.BlockSpec((EMBED_DIM, VBLK), lambda i: (0, i))],
        out_specs=pl.BlockSpec((VBLK // 4, 128), lambda i: (i, 0)),
        out_shape=jax.ShapeDtypeStruct((NVBLK * VBLK // 4, 128), jnp.float32),
    )(token_table.T)
    tok_lin = tok_lin2.reshape(NVBLK * VBLK, EMBED_DIM)

    x = x.astype(jnp.int32)
    q = x % VBLK
    xr = (x - q) + 4 * (q % QB) + q // QB

    xt = (xr.T.reshape(SEQ, BT, 4, 32)
          .transpose(0, 1, 3, 2)
          .reshape(NSLAB, 128))
    mesh = plsc.VectorSubcoreMesh(core_axis_name="c", subcore_axis_name="s")
    gather = pl.kernel(
        _gather_body,
        out_type=jax.ShapeDtypeStruct((NSLAB, 128, EMBED_DIM), jnp.float32),
        mesh=mesh,
        scratch_types=[
            pltpu.VMEM((SPW, 128), jnp.int32),
            pltpu.VMEM((MAXLEN, EMBED_DIM), jnp.float32),
            [pltpu.VMEM((128, EMBED_DIM), jnp.float32) for _ in range(NBUF)],
            [pltpu.VMEM((128, EMBED_DIM), jnp.float32) for _ in range(NBUF)],
            [pltpu.SemaphoreType.DMA for _ in range(NBUF)],
            [pltpu.SemaphoreType.DMA for _ in range(NBUF)],
        ],
        compiler_params=pltpu.CompilerParams(use_tc_tiling_on_sc=False),
    )
    inter = gather(xt, tok_lin, position_table)
    inter2 = inter.reshape(SEQ * BATCH * EMBED_DIM // 128, 128)

    p5 = pl.pallas_call(
        _relayout_body,
        grid=(SEQ // SBLK,),
        in_specs=[pl.BlockSpec((SBLK * BATCH * EMBED_DIM // 128, 128),
                               lambda s: (s, 0))],
        out_specs=pl.BlockSpec((SBLK, DG, BT, 8, 128),
                               lambda s: (s, 0, 0, 0, 0)),
        out_shape=jax.ShapeDtypeStruct((SEQ, DG, BT, 8, 128), jnp.float32),
    )(inter2)

    return (p5.transpose(2, 4, 0, 1, 3).reshape(BATCH, SEQ, EMBED_DIM))


# speedup vs baseline: 4.1543x; 1.4578x over previous
import jax
import jax.numpy as jnp
from jax import lax
from jax.experimental import pallas as pl
from jax.experimental.pallas import tpu as pltpu
from jax.experimental.pallas import tpu_sc as plsc

NUM_VOCAB = 1000000
MAXLEN = 200
EMBED_DIM = 32
BATCH = 4096
SEQ = 200

NC = 2
NS = 16
NW = NC * NS
BT = BATCH // 128
NSLAB = SEQ * BT
SPW = NSLAB // NW
DG = EMBED_DIM // 8
LANES = 16
NBUF = 4


def _gather_body(x_hbm, tok_hbm, pos_hbm, out_hbm,
                 idx_v, pos_v, gbufs, wbufs, gsems, wsems):
    c = lax.axis_index("c")
    s_ax = lax.axis_index("s")
    wid = s_ax * NC + c
    slab0 = wid * SPW

    pltpu.sync_copy(pos_hbm, pos_v)
    pltpu.sync_copy(x_hbm.at[pl.ds(slab0, SPW)], idx_v)

    def start_gather(k, b):
        pltpu.async_copy(tok_hbm.at[idx_v.at[k]], gbufs[b], gsems[b])

    def wait_gather(b):
        pltpu.make_async_copy(
            tok_hbm.at[pl.ds(0, 128)], gbufs[b], gsems[b]).wait()

    def start_wb(k, b):
        pltpu.async_copy(wbufs[b], out_hbm.at[slab0 + k], wsems[b])

    def wait_wb(b):
        pltpu.make_async_copy(wbufs[b], out_hbm.at[0], wsems[b]).wait()

    for b in range(NBUF):
        start_gather(b, b)

    @pl.loop(0, SPW, step=NBUF)
    def _(g):
        for b in range(NBUF):
            k = g + b
            s = (slab0 + k) // BT
            wait_gather(b)

            @pl.when(g > 0)
            def _():
                wait_wb(b)

            gbuf, wbuf = gbufs[b], wbufs[b]
            p0 = pos_v[s, pl.ds(0, LANES)]
            p1 = pos_v[s, pl.ds(LANES, LANES)]

            @pl.loop(0, 128)
            def _(r):
                wbuf[r, pl.ds(0, LANES)] = gbuf[r, pl.ds(0, LANES)] + p0
                wbuf[r, pl.ds(LANES, LANES)] = (
                    gbuf[r, pl.ds(LANES, LANES)] + p1)

            @pl.when(g < SPW - NBUF)
            def _():
                start_gather(k + NBUF, b)

            start_wb(k, b)

    for b in range(NBUF):
        wait_wb(b)


VBLK = 32768
NVBLK = -(-NUM_VOCAB // VBLK)
QB = VBLK // 4
SBLK = 8


def _conv_body(in_ref, out_ref):
    g = in_ref[...]
    j = jnp.concatenate(
        [g[:, m * QB:(m + 1) * QB] for m in range(4)], axis=0)
    out_ref[...] = j.T


def _relayout_body(in_ref, out_ref):
    t = in_ref[...].T
    for sl in range(SBLK):
        for dg in range(DG):
            for bt in range(BT):
                c = sl * 1024 + bt * 32
                out_ref[sl, dg, bt] = jnp.concatenate(
                    [t[m * 32 + dg * 8:m * 32 + dg * 8 + 8,
                       c:c + 32] for m in range(4)],
                    axis=-1)


def kernel(x, token_table, position_table):
    tok_lin2 = pl.pallas_call(
        _conv_body,
        grid=(NVBLK,),
        in_specs=[pl.BlockSpec((EMBED_DIM, VBLK), lambda i: (0, i))],
        out_specs=pl.BlockSpec((VBLK // 4, 128), lambda i: (i, 0)),
        out_shape=jax.ShapeDtypeStruct((NVBLK * VBLK // 4, 128), jnp.float32),
    )(token_table.T)
    tok_lin = tok_lin2.reshape(NVBLK * VBLK, EMBED_DIM)

    x = x.astype(jnp.int32)
    q = x % VBLK
    xr = (x - q) + 4 * (q % QB) + q // QB

    xt = (xr.T.reshape(SEQ, BT, 4, 32)
          .transpose(0, 1, 3, 2)
          .reshape(NSLAB, 128))
    mesh = plsc.VectorSubcoreMesh(core_axis_name="c", subcore_axis_name="s")
    gather = pl.kernel(
        _gather_body,
        out_type=jax.ShapeDtypeStruct((NSLAB, 128, EMBED_DIM), jnp.float32),
        mesh=mesh,
        scratch_types=[
            pltpu.VMEM((SPW, 128), jnp.int32),
            pltpu.VMEM((MAXLEN, EMBED_DIM), jnp.float32),
            [pltpu.VMEM((128, EMBED_DIM), jnp.float32) for _ in range(NBUF)],
            [pltpu.VMEM((128, EMBED_DIM), jnp.float32) for _ in range(NBUF)],
            [pltpu.SemaphoreType.DMA for _ in range(NBUF)],
            [pltpu.SemaphoreType.DMA for _ in range(NBUF)],
        ],
        compiler_params=pltpu.CompilerParams(use_tc_tiling_on_sc=False),
    )
    inter = gather(xt, tok_lin, position_table)
    inter2 = inter.reshape(SEQ * BATCH * EMBED_DIM // 128, 128)

    p5 = pl.pallas_call(
        _relayout_body,
        grid=(SEQ // SBLK,),
        in_specs=[pl.BlockSpec((SBLK * BATCH * EMBED_DIM // 128, 128),
                               lambda s: (s, 0))],
        out_specs=pl.BlockSpec((SBLK, DG, BT, 8, 128),
                               lambda s: (s, 0, 0, 0, 0)),
        out_shape=jax.ShapeDtypeStruct((SEQ, DG, BT, 8, 128), jnp.float32),
    )(inter2)

    return (p5.transpose(2, 4, 0, 1, 3).reshape(BATCH, SEQ, EMBED_DIM))


# speedup vs baseline: 4.3550x; 1.0483x over previous
import jax
import jax.numpy as jnp
from jax import lax
from jax.experimental import pallas as pl
from jax.experimental.pallas import tpu as pltpu
from jax.experimental.pallas import tpu_sc as plsc

NUM_VOCAB = 1000000
MAXLEN = 200
EMBED_DIM = 32
BATCH = 4096
SEQ = 200

NC = 2
NS = 16
NW = NC * NS
BT = BATCH // 128
NSLAB = SEQ * BT
SPW = NSLAB // NW
DG = EMBED_DIM // 8
LANES = 16
NBUF = 4


def _gather_body(x_hbm, tok_hbm, pos_hbm, out_hbm,
                 idx_v, pos_v, gbufs, wbufs, gsems, wsems):
    c = lax.axis_index("c")
    s_ax = lax.axis_index("s")
    wid = s_ax * NC + c
    slab0 = wid * SPW

    pltpu.sync_copy(pos_hbm, pos_v)
    pltpu.sync_copy(x_hbm.at[pl.ds(slab0, SPW)], idx_v)

    def start_gather(k, b):
        pltpu.async_copy(tok_hbm.at[idx_v.at[k]], gbufs[b], gsems[b])

    def wait_gather(b):
        pltpu.make_async_copy(
            tok_hbm.at[pl.ds(0, 128)], gbufs[b], gsems[b]).wait()

    def start_wb(k, b):
        pltpu.async_copy(wbufs[b], out_hbm.at[slab0 + k], wsems[b])

    def wait_wb(b):
        pltpu.make_async_copy(wbufs[b], out_hbm.at[0], wsems[b]).wait()

    for b in range(NBUF):
        start_gather(b, b)

    @pl.loop(0, SPW, step=NBUF)
    def _(g):
        for b in range(NBUF):
            k = g + b
            s = (slab0 + k) // BT
            wait_gather(b)

            @pl.when(g > 0)
            def _():
                wait_wb(b)

            gbuf, wbuf = gbufs[b], wbufs[b]
            p0 = pos_v[s, pl.ds(0, LANES)]
            p1 = pos_v[s, pl.ds(LANES, LANES)]

            @pl.loop(0, 128, step=4)
            def _(r):
                for u in range(4):
                    wbuf[r + u, pl.ds(0, LANES)] = (
                        gbuf[r + u, pl.ds(0, LANES)] + p0)
                    wbuf[r + u, pl.ds(LANES, LANES)] = (
                        gbuf[r + u, pl.ds(LANES, LANES)] + p1)

            @pl.when(g < SPW - NBUF)
            def _():
                start_gather(k + NBUF, b)

            start_wb(k, b)

    for b in range(NBUF):
        wait_wb(b)


VBLK = 32768
NVBLK = -(-NUM_VOCAB // VBLK)
QB = VBLK // 4
SBLK = 8


def _conv_body(in_ref, out_ref):
    g = in_ref[...]
    j = jnp.concatenate(
        [g[:, m * QB:(m + 1) * QB] for m in range(4)], axis=0)
    out_ref[...] = j.T


def _relayout_body(in_ref, out_ref):
    t = in_ref[...].T
    for sl in range(SBLK):
        for dg in range(DG):
            for bt in range(BT):
                c = sl * 1024 + bt * 32
                out_ref[sl, dg, bt] = jnp.concatenate(
                    [t[m * 32 + dg * 8:m * 32 + dg * 8 + 8,
                       c:c + 32] for m in range(4)],
                    axis=-1)


def kernel(x, token_table, position_table):
    tok_lin2 = pl.pallas_call(
        _conv_body,
        grid=(NVBLK,),
        in_specs=[pl.BlockSpec((EMBED_DIM, VBLK), lambda i: (0, i))],
        out_specs=pl.BlockSpec((VBLK // 4, 128), lambda i: (i, 0)),
        out_shape=jax.ShapeDtypeStruct((NVBLK * VBLK // 4, 128), jnp.float32),
    )(token_table.T)
    tok_lin = tok_lin2.reshape(NVBLK * VBLK, EMBED_DIM)

    x = x.astype(jnp.int32)
    q = x % VBLK
    xr = (x - q) + 4 * (q % QB) + q // QB

    xt = (xr.T.reshape(SEQ, BT, 4, 32)
          .transpose(0, 1, 3, 2)
          .reshape(NSLAB, 128))
    mesh = plsc.VectorSubcoreMesh(core_axis_name="c", subcore_axis_name="s")
    gather = pl.kernel(
        _gather_body,
        out_type=jax.ShapeDtypeStruct((NSLAB, 128, EMBED_DIM), jnp.float32),
        mesh=mesh,
        scratch_types=[
            pltpu.VMEM((SPW, 128), jnp.int32),
            pltpu.VMEM((MAXLEN, EMBED_DIM), jnp.float32),
            [pltpu.VMEM((128, EMBED_DIM), jnp.float32) for _ in range(NBUF)],
            [pltpu.VMEM((128, EMBED_DIM), jnp.float32) for _ in range(NBUF)],
            [pltpu.SemaphoreType.DMA for _ in range(NBUF)],
            [pltpu.SemaphoreType.DMA for _ in range(NBUF)],
        ],
        compiler_params=pltpu.CompilerParams(use_tc_tiling_on_sc=False),
    )
    inter = gather(xt, tok_lin, position_table)
    inter2 = inter.reshape(SEQ * BATCH * EMBED_DIM // 128, 128)

    p5 = pl.pallas_call(
        _relayout_body,
        grid=(SEQ // SBLK,),
        in_specs=[pl.BlockSpec((SBLK * BATCH * EMBED_DIM // 128, 128),
                               lambda s: (s, 0))],
        out_specs=pl.BlockSpec((SBLK, DG, BT, 8, 128),
                               lambda s: (s, 0, 0, 0, 0)),
        out_shape=jax.ShapeDtypeStruct((SEQ, DG, BT, 8, 128), jnp.float32),
    )(inter2)

    return (p5.transpose(2, 4, 0, 1, 3).reshape(BATCH, SEQ, EMBED_DIM))


# speedup vs baseline: 4.3837x; 1.0066x over previous
import jax
import jax.numpy as jnp
from jax import lax
from jax.experimental import pallas as pl
from jax.experimental.pallas import tpu as pltpu
from jax.experimental.pallas import tpu_sc as plsc

NUM_VOCAB = 1000000
MAXLEN = 200
EMBED_DIM = 32
BATCH = 4096
SEQ = 200

NC = 2
NS = 16
NW = NC * NS
BT = BATCH // 128
NSLAB = SEQ * BT
SPW = NSLAB // NW
DG = EMBED_DIM // 8
LANES = 16
NBUF = 5


def _gather_body(x_hbm, tok_hbm, pos_hbm, out_hbm,
                 idx_v, pos_v, gbufs, wbufs, gsems, wsems):
    c = lax.axis_index("c")
    s_ax = lax.axis_index("s")
    wid = s_ax * NC + c
    slab0 = wid * SPW

    pltpu.sync_copy(pos_hbm, pos_v)
    pltpu.sync_copy(x_hbm.at[pl.ds(slab0, SPW)], idx_v)

    def start_gather(k, b):
        pltpu.async_copy(tok_hbm.at[idx_v.at[k]], gbufs[b], gsems[b])

    def wait_gather(b):
        pltpu.make_async_copy(
            tok_hbm.at[pl.ds(0, 128)], gbufs[b], gsems[b]).wait()

    def start_wb(k, b):
        pltpu.async_copy(wbufs[b], out_hbm.at[slab0 + k], wsems[b])

    def wait_wb(b):
        pltpu.make_async_copy(wbufs[b], out_hbm.at[0], wsems[b]).wait()

    for b in range(NBUF):
        start_gather(b, b)

    @pl.loop(0, SPW, step=NBUF)
    def _(g):
        for b in range(NBUF):
            k = g + b
            s = (slab0 + k) // BT
            wait_gather(b)

            @pl.when(g > 0)
            def _():
                wait_wb(b)

            gbuf, wbuf = gbufs[b], wbufs[b]
            p0 = pos_v[s, pl.ds(0, LANES)]
            p1 = pos_v[s, pl.ds(LANES, LANES)]

            @pl.loop(0, 128, step=4)
            def _(r):
                for u in range(4):
                    wbuf[r + u, pl.ds(0, LANES)] = (
                        gbuf[r + u, pl.ds(0, LANES)] + p0)
                    wbuf[r + u, pl.ds(LANES, LANES)] = (
                        gbuf[r + u, pl.ds(LANES, LANES)] + p1)

            @pl.when(g < SPW - NBUF)
            def _():
                start_gather(k + NBUF, b)

            start_wb(k, b)

    for b in range(NBUF):
        wait_wb(b)


VBLK = 32768
NVBLK = -(-NUM_VOCAB // VBLK)
QB = VBLK // 4
SBLK = 8


def _conv_body(in_ref, out_ref):
    g = in_ref[...]
    j = jnp.concatenate(
        [g[:, m * QB:(m + 1) * QB] for m in range(4)], axis=0)
    out_ref[...] = j.T


def _relayout_body(in_ref, out_ref):
    t = in_ref[...].T
    for sl in range(SBLK):
        for dg in range(DG):
            for bt in range(BT):
                c = sl * 1024 + bt * 32
                out_ref[sl, dg, bt] = jnp.concatenate(
                    [t[m * 32 + dg * 8:m * 32 + dg * 8 + 8,
                       c:c + 32] for m in range(4)],
                    axis=-1)


def kernel(x, token_table, position_table):
    tok_lin2 = pl.pallas_call(
        _conv_body,
        grid=(NVBLK,),
        in_specs=[pl.BlockSpec((EMBED_DIM, VBLK), lambda i: (0, i))],
        out_specs=pl.BlockSpec((VBLK // 4, 128), lambda i: (i, 0)),
        out_shape=jax.ShapeDtypeStruct((NVBLK * VBLK // 4, 128), jnp.float32),
    )(token_table.T)
    tok_lin = tok_lin2.reshape(NVBLK * VBLK, EMBED_DIM)

    x = x.astype(jnp.int32)
    q = x % VBLK
    xr = (x - q) + 4 * (q % QB) + q // QB

    xt = (xr.T.reshape(SEQ, BT, 4, 32)
          .transpose(0, 1, 3, 2)
          .reshape(NSLAB, 128))
    mesh = plsc.VectorSubcoreMesh(core_axis_name="c", subcore_axis_name="s")
    gather = pl.kernel(
        _gather_body,
        out_type=jax.ShapeDtypeStruct((NSLAB, 128, EMBED_DIM), jnp.float32),
        mesh=mesh,
        scratch_types=[
            pltpu.VMEM((SPW, 128), jnp.int32),
            pltpu.VMEM((MAXLEN, EMBED_DIM), jnp.float32),
            [pltpu.VMEM((128, EMBED_DIM), jnp.float32) for _ in range(NBUF)],
            [pltpu.VMEM((128, EMBED_DIM), jnp.float32) for _ in range(NBUF)],
            [pltpu.SemaphoreType.DMA for _ in range(NBUF)],
            [pltpu.SemaphoreType.DMA for _ in range(NBUF)],
        ],
        compiler_params=pltpu.CompilerParams(use_tc_tiling_on_sc=False),
    )
    inter = gather(xt, tok_lin, position_table)
    inter2 = inter.reshape(SEQ * BATCH * EMBED_DIM // 128, 128)

    p5 = pl.pallas_call(
        _relayout_body,
        grid=(SEQ // SBLK,),
        in_specs=[pl.BlockSpec((SBLK * BATCH * EMBED_DIM // 128, 128),
                               lambda s: (s, 0))],
        out_specs=pl.BlockSpec((SBLK, DG, BT, 8, 128),
                               lambda s: (s, 0, 0, 0, 0)),
        out_shape=jax.ShapeDtypeStruct((SEQ, DG, BT, 8, 128), jnp.float32),
    )(inter2)

    return (p5.transpose(2, 4, 0, 1, 3).reshape(BATCH, SEQ, EMBED_DIM))
